# baseline (speedup 1.0000x reference)
import functools

import jax
import jax.numpy as jnp
from jax import lax
from jax.experimental import pallas as pl
from jax.experimental.pallas import tpu as pltpu
from jax.experimental.pallas import tpu_sc as plsc

DIM = 128
LANES = 16
CHUNK = 128
RAMP = (32, 96)


def _make_kernel(batch):
    info = plsc.get_sparse_core_info()
    nc, ns = info.num_cores, info.num_subcores
    nw = nc * ns
    bpw = batch // nw

    chunks = []
    off = 0
    for sz in RAMP:
        chunks.append((off, sz))
        off += sz
    while off < bpw:
        chunks.append((off, CHUNK))
        off += CHUNK
    assert off == bpw

    mesh = plsc.VectorSubcoreMesh(core_axis_name="c", subcore_axis_name="s")

    @functools.partial(
        pl.kernel,
        mesh=mesh,
        out_type=jax.ShapeDtypeStruct((batch,), jnp.float32),
        compiler_params=pltpu.CompilerParams(
            needs_layout_passes=False, skip_device_barrier=True),
        scratch_types=[
            pltpu.VMEM((bpw,), jnp.int32),
            pltpu.VMEM((bpw,), jnp.int32),
            pltpu.VMEM((3, CHUNK, DIM), jnp.float32),
            pltpu.VMEM((3, CHUNK, DIM), jnp.float32),
            pltpu.VMEM((bpw,), jnp.float32),
            pltpu.SemaphoreType.DMA,
            pltpu.SemaphoreType.DMA,
            pltpu.SemaphoreType.DMA,
            pltpu.SemaphoreType.DMA,
            pltpu.SemaphoreType.DMA,
            pltpu.SemaphoreType.DMA,
            pltpu.SemaphoreType.DMA,
            pltpu.SemaphoreType.DMA,
            pltpu.SemaphoreType.DMA,
        ],
    )
    def two_tower(u_idx_hbm, i_idx_hbm, user_hbm, item_hbm, out_hbm,
                  uidx_v, iidx_v, u_rows, i_rows, out_v,
                  uisem, iisem, usem0, usem1, usem2, isem0, isem1, isem2,
                  osem):
        wid = lax.axis_index("s") * nc + lax.axis_index("c")
        base = wid * bpw
        hui = pltpu.async_copy(u_idx_hbm.at[pl.ds(base, bpw)], uidx_v, uisem)
        hii = pltpu.async_copy(i_idx_hbm.at[pl.ds(base, bpw)], iidx_v, iisem)
        hui.wait()
        hii.wait()
        last_lane = lax.iota(jnp.int32, LANES) == (LANES - 1)
        usems = (usem0, usem1, usem2)
        isems = (isem0, isem1, isem2)

        def start_gathers(c):
            b = c % 3
            off, sz = chunks[c]
            hu = pltpu.async_copy(
                user_hbm.at[uidx_v.at[pl.ds(off, sz)]],
                u_rows.at[b, pl.ds(0, sz)], usems[b])
            hi = pltpu.async_copy(
                item_hbm.at[iidx_v.at[pl.ds(off, sz)]],
                i_rows.at[b, pl.ds(0, sz)], isems[b])
            return hu, hi

        def start_gathers_split(c):
            b = c % 3
            off, sz = chunks[c]
            h1 = sz // 2
            hs = []
            for (tbl, idxv, rows, sem) in (
                    (user_hbm, uidx_v, u_rows, usems[b]),
                    (item_hbm, iidx_v, i_rows, isems[b])):
                hs.append(pltpu.async_copy(
                    tbl.at[idxv.at[pl.ds(off, h1)]],
                    rows.at[b, pl.ds(0, h1)], sem))
                hs.append(pltpu.async_copy(
                    tbl.at[idxv.at[pl.ds(off + h1, sz - h1)]],
                    rows.at[b, pl.ds(h1, sz - h1)], sem))
            return hs

        handles = [start_gathers_split(0), start_gathers_split(1)] + \
            [None] * (len(chunks) - 2)
        drains = []
        for c, (off, sz) in enumerate(chunks):
            b = c % 3
            for h in handles[c]:
                h.wait()
            if c + 2 < len(chunks):
                handles[c + 2] = start_gathers_split(c + 2)
            ub = u_rows.at[b]
            ib = i_rows.at[b]

            def load_row(row):
                return [(ub[row, pl.ds(cc * LANES, LANES)],
                         ib[row, pl.ds(cc * LANES, LANES)])
                        for cc in range(DIM // LANES)]

            def group_body(g, _):
                idx_base = jnp.full((LANES,), off, jnp.int32) + g * LANES
                cur = load_row(g * LANES)
                for r in range(LANES):
                    nxt = load_row(g * LANES + r + 1) if r + 1 < LANES \
                        else None
                    p0 = cur[0][0] * cur[0][1]
                    p1 = cur[1][0] * cur[1][1]
                    for cc in range(2, DIM // LANES, 2):
                        p0 = p0 + cur[cc][0] * cur[cc][1]
                        p1 = p1 + cur[cc + 1][0] * cur[cc + 1][1]
                    csum = plsc.cumsum(p0 + p1)
                    plsc.store_scatter(
                        out_v, [idx_base + r], csum, mask=last_lane)
                    cur = nxt
                return 0

            lax.fori_loop(0, sz // LANES, group_body, 0)
            drains.append(pltpu.async_copy(
                out_v.at[pl.ds(off, sz)],
                out_hbm.at[pl.ds(base + off, sz)], osem))

        for d in drains:
            d.wait()

    return two_tower


@jax.jit
def kernel(u_idx, i_idx, user_emb, item_emb):
    return _make_kernel(u_idx.shape[0])(u_idx, i_idx, user_emb, item_emb)


# speedup vs baseline: 1.0060x; 1.0060x over previous
import functools

import jax
import jax.numpy as jnp
from jax import lax
from jax.experimental import pallas as pl
from jax.experimental.pallas import tpu as pltpu
from jax.experimental.pallas import tpu_sc as plsc

DIM = 128
LANES = 16
CHUNK = 128
RAMP = (32, 96)


def _make_kernel(batch):
    info = plsc.get_sparse_core_info()
    nc, ns = info.num_cores, info.num_subcores
    nw = nc * ns
    bpw = batch // nw

    chunks = []
    off = 0
    for sz in RAMP:
        chunks.append((off, sz))
        off += sz
    while off < bpw:
        chunks.append((off, CHUNK))
        off += CHUNK
    assert off == bpw

    mesh = plsc.VectorSubcoreMesh(core_axis_name="c", subcore_axis_name="s")

    @functools.partial(
        pl.kernel,
        mesh=mesh,
        out_type=jax.ShapeDtypeStruct((batch,), jnp.float32),
        compiler_params=pltpu.CompilerParams(
            needs_layout_passes=False, skip_device_barrier=True),
        scratch_types=[
            pltpu.VMEM((bpw,), jnp.int32),
            pltpu.VMEM((bpw,), jnp.int32),
            pltpu.VMEM((3, CHUNK, DIM), jnp.float32),
            pltpu.VMEM((3, CHUNK, DIM), jnp.float32),
            pltpu.VMEM((bpw,), jnp.float32),
            pltpu.SemaphoreType.DMA,
            pltpu.SemaphoreType.DMA,
            pltpu.SemaphoreType.DMA,
            pltpu.SemaphoreType.DMA,
            pltpu.SemaphoreType.DMA,
            pltpu.SemaphoreType.DMA,
            pltpu.SemaphoreType.DMA,
            pltpu.SemaphoreType.DMA,
            pltpu.SemaphoreType.DMA,
        ],
    )
    def two_tower(u_idx_hbm, i_idx_hbm, user_hbm, item_hbm, out_hbm,
                  uidx_v, iidx_v, u_rows, i_rows, out_v,
                  uisem, iisem, usem0, usem1, usem2, isem0, isem1, isem2,
                  osem):
        wid = lax.axis_index("s") * nc + lax.axis_index("c")
        base = wid * bpw
        hui = pltpu.async_copy(u_idx_hbm.at[pl.ds(base, bpw)], uidx_v, uisem)
        hii = pltpu.async_copy(i_idx_hbm.at[pl.ds(base, bpw)], iidx_v, iisem)
        hui.wait()
        hii.wait()
        last_lane = lax.iota(jnp.int32, LANES) == (LANES - 1)
        usems = (usem0, usem1, usem2)
        isems = (isem0, isem1, isem2)

        def start_gathers(c):
            b = c % 3
            off, sz = chunks[c]
            hu = pltpu.async_copy(
                user_hbm.at[uidx_v.at[pl.ds(off, sz)]],
                u_rows.at[b, pl.ds(0, sz)], usems[b])
            hi = pltpu.async_copy(
                item_hbm.at[iidx_v.at[pl.ds(off, sz)]],
                i_rows.at[b, pl.ds(0, sz)], isems[b])
            return hu, hi

        handles = [start_gathers(0), start_gathers(1)] + \
            [None] * (len(chunks) - 2)
        drains = []
        for c, (off, sz) in enumerate(chunks):
            b = c % 3
            handles[c][0].wait()
            handles[c][1].wait()
            if c + 2 < len(chunks):
                handles[c + 2] = start_gathers(c + 2)
            ub = u_rows.at[b]
            ib = i_rows.at[b]

            def load_row(row):
                return [(ub[row, pl.ds(cc * LANES, LANES)],
                         ib[row, pl.ds(cc * LANES, LANES)])
                        for cc in range(DIM // LANES)]

            def group_body(g, _):
                idx_base = jnp.full((LANES,), off, jnp.int32) + g * LANES
                cur = load_row(g * LANES)
                for r in range(LANES):
                    nxt = load_row(g * LANES + r + 1) if r + 1 < LANES \
                        else None
                    p0 = cur[0][0] * cur[0][1]
                    p1 = cur[1][0] * cur[1][1]
                    for cc in range(2, DIM // LANES, 2):
                        p0 = p0 + cur[cc][0] * cur[cc][1]
                        p1 = p1 + cur[cc + 1][0] * cur[cc + 1][1]
                    csum = plsc.cumsum(p0 + p1)
                    plsc.store_scatter(
                        out_v, [idx_base + r], csum, mask=last_lane)
                    cur = nxt
                return 0

            lax.fori_loop(0, sz // LANES, group_body, 0)
            drains.append(pltpu.async_copy(
                out_v.at[pl.ds(off, sz)],
                out_hbm.at[pl.ds(base + off, sz)], osem))

        for d in drains:
            d.wait()

    return two_tower


@jax.jit
def kernel(u_idx, i_idx, user_emb, item_emb):
    return _make_kernel(u_idx.shape[0])(u_idx, i_idx, user_emb, item_emb)
